# baseline (speedup 1.0000x reference)
import jax
import jax.numpy as jnp
from jax import lax
from jax.experimental import pallas as pl
from jax.experimental.pallas import tpu as pltpu
from jax.experimental.pallas import tpu_sc as plsc

B = 4096
L = 50
NF = 26
D = 64
NC = 2
NS = 16
NW = NC * NS
RPW = B // NW
P = 2
NPH = RPW // P
NBUF = 8
SROWS = P * L
CROWS = P * NF


def _make_kernel():
    mesh = plsc.VectorSubcoreMesh(core_axis_name="c", subcore_axis_name="s")

    def body(seq_ids_hbm, ctx_ids_hbm, item_hbm, ctx_table_hbm, out_hbm,
             seq_idx_v, ctx_idx_v, rows_v, acc_v, sem):
        cid = lax.axis_index("c")
        sid = lax.axis_index("s")
        wid = sid * NC + cid
        base = wid * RPW

        pltpu.sync_copy(seq_ids_hbm.at[wid], seq_idx_v)
        pltpu.sync_copy(ctx_ids_hbm.at[wid], ctx_idx_v)

        inv_l = jnp.full((16,), 1.0 / L, jnp.float32)

        streams = [
            (item_hbm, seq_idx_v, L, SROWS, 0, inv_l),
            (ctx_table_hbm, ctx_idx_v, NF, CROWS, RPW, None),
        ]

        def run_stream(st):
            tbl, idx_v, spr, rpp, aoff, scale = streams[st]

            def gather(p, b):
                pltpu.async_copy(tbl.at[idx_v.at[p]],
                                 rows_v.at[b, pl.ds(0, rpp)], sem.at[b])

            def wait_gather(p, b):
                pltpu.make_async_copy(tbl.at[idx_v.at[p]],
                                      rows_v.at[b, pl.ds(0, rpp)],
                                      sem.at[b]).wait()

            for b in range(NBUF):
                gather(b, b)

            @pl.loop(0, NPH, step=NBUF)
            def _(p):
                for b in range(NBUF):
                    pe = p + b
                    wait_gather(pe, b)

                    @pl.loop(0, P)
                    def _(s):
                        r0 = s * spr
                        for j in range(D // 16):
                            sl = pl.ds(j * 16, 16)
                            acc = rows_v[b, r0, sl]
                            for r in range(1, spr):
                                acc = acc + rows_v[b, r0 + r, sl]
                            if scale is not None:
                                acc = acc * scale
                            acc_v[aoff + pe * P + s, sl] = acc

                    @pl.when(pe + NBUF < NPH)
                    def _():
                        gather(pe + NBUF, b)

        run_stream(0)
        run_stream(1)

        pltpu.sync_copy(acc_v.at[pl.ds(0, RPW)], out_hbm.at[0, pl.ds(base, RPW)])
        pltpu.sync_copy(acc_v.at[pl.ds(RPW, RPW)], out_hbm.at[1, pl.ds(base, RPW)])

    return pl.kernel(
        body,
        out_type=jax.ShapeDtypeStruct((2, B, D), jnp.float32),
        mesh=mesh,
        compiler_params=pltpu.CompilerParams(use_tc_tiling_on_sc=False),
        scratch_types=[
            pltpu.VMEM((NPH, SROWS), jnp.int32),
            pltpu.VMEM((NPH, CROWS), jnp.int32),
            pltpu.VMEM((NBUF, SROWS, D), jnp.float32),
            pltpu.VMEM((2 * RPW, D), jnp.float32),
            pltpu.SemaphoreType.DMA((NBUF,)),
        ],
    )


_sc_kernel = _make_kernel()


def kernel(seq_item_ids, context_ids, item_table, context_table):
    seq_ids = seq_item_ids.reshape(NW, NPH, SROWS)
    ctx_ids = context_ids.reshape(NW, NPH, CROWS)
    out = _sc_kernel(seq_ids, ctx_ids, item_table, context_table)
    return jnp.concatenate([out[0], out[1]], axis=-1)


# speedup vs baseline: 1.0542x; 1.0542x over previous
import jax
import jax.numpy as jnp
from jax import lax
from jax.experimental import pallas as pl
from jax.experimental.pallas import tpu as pltpu
from jax.experimental.pallas import tpu_sc as plsc

B = 4096
L = 50
NF = 26
D = 64
NC = 2
NS = 16
NW = NC * NS
RPW = B // NW
CH = 128
SEQ_CHUNKS = RPW * L // CH
CTX_CHUNKS = RPW * NF // CH
NB = 10


def _make_kernel():
    mesh = plsc.VectorSubcoreMesh(core_axis_name="c", subcore_axis_name="s")

    def body(seq_ids_hbm, ctx_ids_hbm, pat_seq_hbm, pat_ctx_hbm,
             item_hbm, ctx_table_hbm, out_hbm,
             seq_idx_v, ctx_idx_v, pat_seq_v, pat_ctx_v,
             rows_v, work_v, acc_sh, sem_g, sem_s):
        cid = lax.axis_index("c")
        sid = lax.axis_index("s")
        wid = sid * NC + cid
        base = wid * RPW

        pltpu.sync_copy(seq_ids_hbm.at[wid], seq_idx_v)
        pltpu.sync_copy(ctx_ids_hbm.at[wid], ctx_idx_v)
        pltpu.sync_copy(pat_seq_hbm.at[sid], pat_seq_v)
        pltpu.sync_copy(pat_ctx_hbm.at[sid], pat_ctx_v)

        zero = jnp.zeros((16,), jnp.float32)

        @pl.loop(0, RPW)
        def _(i):
            for j in range(D // 16):
                work_v[i, pl.ds(j * 16, 16)] = zero

        pltpu.sync_copy(work_v, acc_sh.at[pl.ds(sid * RPW, RPW)])
        pltpu.sync_copy(work_v, acc_sh.at[pl.ds(NS * RPW + sid * RPW, RPW)])

        chunks = ([(item_hbm, seq_idx_v, pat_seq_v, c) for c in range(SEQ_CHUNKS)]
                  + [(ctx_table_hbm, ctx_idx_v, pat_ctx_v, c) for c in range(CTX_CHUNKS)])
        tot = len(chunks)
        lag = NB - 1
        g_descs = [None] * tot
        s_descs = [None] * tot
        s_waited = [False] * tot

        def start_gather(t):
            tbl, idx_v, _, c = chunks[t]
            return pltpu.async_copy(tbl.at[idx_v.at[c]], rows_v.at[t % NB],
                                    sem_g.at[t % NB])

        def start_scatter(t):
            _, _, pat_v, c = chunks[t]
            return pltpu.async_copy(rows_v.at[t % NB], acc_sh.at[pat_v.at[c]],
                                    sem_s.at[t % NB], add=True)

        for t in range(tot + lag):
            if t < tot:
                if t >= NB:
                    s_descs[t - NB].wait()
                    s_waited[t - NB] = True
                g_descs[t] = start_gather(t)
            p = t - lag
            if 0 <= p < tot:
                g_descs[p].wait()
                s_descs[p] = start_scatter(p)
        for t in range(tot):
            if not s_waited[t]:
                s_descs[t].wait()

        pltpu.sync_copy(acc_sh.at[pl.ds(sid * RPW, RPW)], work_v)
        inv_l = jnp.full((16,), 1.0 / L, jnp.float32)

        @pl.loop(0, RPW)
        def _(i):
            for j in range(D // 16):
                sl = pl.ds(j * 16, 16)
                work_v[i, sl] = work_v[i, sl] * inv_l

        pltpu.sync_copy(work_v, out_hbm.at[0, pl.ds(base, RPW)])
        pltpu.sync_copy(acc_sh.at[pl.ds(NS * RPW + sid * RPW, RPW)], rows_v.at[0])
        pltpu.sync_copy(rows_v.at[0], out_hbm.at[1, pl.ds(base, RPW)])

    return pl.kernel(
        body,
        out_type=jax.ShapeDtypeStruct((2, B, D), jnp.float32),
        mesh=mesh,
        compiler_params=pltpu.CompilerParams(use_tc_tiling_on_sc=False),
        scratch_types=[
            pltpu.VMEM((SEQ_CHUNKS, CH), jnp.int32),
            pltpu.VMEM((CTX_CHUNKS, CH), jnp.int32),
            pltpu.VMEM((SEQ_CHUNKS, CH), jnp.int32),
            pltpu.VMEM((CTX_CHUNKS, CH), jnp.int32),
            pltpu.VMEM((NB, CH, D), jnp.float32),
            pltpu.VMEM((RPW, D), jnp.float32),
            pltpu.VMEM_SHARED((2 * NS * RPW, D), jnp.float32),
            pltpu.SemaphoreType.DMA((NB,)),
            pltpu.SemaphoreType.DMA((NB,)),
        ],
    )


_sc_kernel = _make_kernel()


def kernel(seq_item_ids, context_ids, item_table, context_table):
    seq_ids = seq_item_ids.reshape(NW, SEQ_CHUNKS, CH)
    ctx_ids = context_ids.reshape(NW, CTX_CHUNKS, CH)

    j_seq = jnp.arange(SEQ_CHUNKS * CH, dtype=jnp.int32) // L
    j_ctx = jnp.arange(CTX_CHUNKS * CH, dtype=jnp.int32) // NF
    s_off = jnp.arange(NS, dtype=jnp.int32)[:, None] * RPW
    pat_seq = (s_off + j_seq[None, :]).reshape(NS, SEQ_CHUNKS, CH)
    pat_ctx = (NS * RPW + s_off + j_ctx[None, :]).reshape(NS, CTX_CHUNKS, CH)

    out = _sc_kernel(seq_ids, ctx_ids, pat_seq, pat_ctx,
                     item_table, context_table)
    return jnp.concatenate([out[0], out[1]], axis=-1)
